# baseline (speedup 1.0000x reference)
import jax
import jax.numpy as jnp
from jax.experimental import pallas as pl

_BQ = 512


def _body(q_ref, k_ref, idx_ref, dist_ref):
    q = q_ref[...]
    k = k_ref[...]
    dots = jax.lax.dot_general(
        q, k, (((1,), (1,)), ((), ())), preferred_element_type=jnp.float32
    )
    k_sq = jnp.sum(k * k, axis=1)[None, :]
    q_sq = jnp.sum(q * q, axis=1, keepdims=True)
    s = q_sq + k_sq - 2.0 * dots
    m = jnp.min(s, axis=1, keepdims=True)
    n_keys = s.shape[1]
    iota = jax.lax.broadcasted_iota(jnp.int32, s.shape, 1)
    idx = jnp.min(jnp.where(s == m, iota, n_keys), axis=1)
    idx_ref[0, 0, :] = idx
    dist_ref[0, 0, :] = jnp.sqrt(jnp.maximum(m[:, 0], 1e-12))


def kernel(queries, keys):
    Q, D = queries.shape
    K, _ = keys.shape
    grid = Q // _BQ
    idx, dist = pl.pallas_call(
        _body,
        grid=(grid,),
        in_specs=[
            pl.BlockSpec((_BQ, D), lambda i: (i, 0)),
            pl.BlockSpec((K, D), lambda i: (0, 0)),
        ],
        out_specs=[
            pl.BlockSpec((1, 1, _BQ), lambda i: (i, 0, 0)),
            pl.BlockSpec((1, 1, _BQ), lambda i: (i, 0, 0)),
        ],
        out_shape=[
            jax.ShapeDtypeStruct((grid, 1, _BQ), jnp.int32),
            jax.ShapeDtypeStruct((grid, 1, _BQ), jnp.float32),
        ],
    )(queries, keys)
    return idx.reshape(Q), dist.reshape(Q)


# speedup vs baseline: 1.3712x; 1.3712x over previous
import jax
import jax.numpy as jnp
from jax.experimental import pallas as pl

_BQ = 512
_CK = 128


def _body(q_ref, k_ref, qsq_ref, ksq_ref, idx_ref, dist_ref):
    q = q_ref[...]
    k = k_ref[...]
    q_sq = qsq_ref[...]
    k_sq = ksq_ref[...]
    n_keys = k.shape[0]
    runm = runi = None
    for c in range(0, n_keys, _CK):
        dots_c = jax.lax.dot_general(
            k[c:c + _CK], q, (((1,), (1,)), ((), ())),
            preferred_element_type=jnp.float32,
        )
        sc = (q_sq + k_sq[c:c + _CK]) - 2.0 * dots_c
        mc = jnp.min(sc, axis=0)
        iota = jax.lax.broadcasted_iota(jnp.int32, sc.shape, 0)
        ic = jnp.min(jnp.where(sc == mc[None, :], iota, _CK), axis=0) + c
        if runm is None:
            runm, runi = mc, ic
        else:
            better = mc < runm
            runi = jnp.where(better, ic, runi)
            runm = jnp.where(better, mc, runm)
    idx_ref[0, 0, :] = runi
    dist_ref[0, 0, :] = jnp.sqrt(jnp.maximum(runm, 1e-12))


def kernel(queries, keys):
    Q, D = queries.shape
    K, _ = keys.shape
    q_sq = jnp.sum(queries * queries, axis=1)[None, :]
    k_sq = jnp.sum(keys * keys, axis=1)[:, None]
    grid = Q // _BQ
    idx, dist = pl.pallas_call(
        _body,
        grid=(grid,),
        in_specs=[
            pl.BlockSpec((_BQ, D), lambda i: (i, 0)),
            pl.BlockSpec((K, D), lambda i: (0, 0)),
            pl.BlockSpec((1, _BQ), lambda i: (0, i)),
            pl.BlockSpec((K, 1), lambda i: (0, 0)),
        ],
        out_specs=[
            pl.BlockSpec((1, 1, _BQ), lambda i: (i, 0, 0)),
            pl.BlockSpec((1, 1, _BQ), lambda i: (i, 0, 0)),
        ],
        out_shape=[
            jax.ShapeDtypeStruct((grid, 1, _BQ), jnp.int32),
            jax.ShapeDtypeStruct((grid, 1, _BQ), jnp.float32),
        ],
    )(queries, keys, q_sq, k_sq)
    return idx.reshape(Q), dist.reshape(Q)


# speedup vs baseline: 1.5202x; 1.1087x over previous
import jax
import jax.numpy as jnp
from jax.experimental import pallas as pl

_BQ = 2048
_CK = 128


def _body(q_ref, k_ref, qsq_ref, ksq_ref, idx_ref, dist_ref):
    q = q_ref[...]
    k = k_ref[...]
    q_sq = qsq_ref[...]
    k_sq = ksq_ref[...]
    n_keys = k.shape[0]
    runm = runi = None
    for c in range(0, n_keys, _CK):
        dots_c = jax.lax.dot_general(
            k[c:c + _CK], q, (((1,), (1,)), ((), ())),
            preferred_element_type=jnp.float32,
        )
        sc = (q_sq + k_sq[c:c + _CK]) - 2.0 * dots_c
        mc = jnp.min(sc, axis=0)
        iota = jax.lax.broadcasted_iota(jnp.int32, sc.shape, 0)
        ic = jnp.min(jnp.where(sc == mc[None, :], iota, _CK), axis=0) + c
        if runm is None:
            runm, runi = mc, ic
        else:
            better = mc < runm
            runi = jnp.where(better, ic, runi)
            runm = jnp.where(better, mc, runm)
    idx_ref[0, 0, :] = runi
    dist_ref[0, 0, :] = jnp.sqrt(jnp.maximum(runm, 1e-12))


def kernel(queries, keys):
    Q, D = queries.shape
    K, _ = keys.shape
    q_sq = jnp.sum(queries * queries, axis=1)[None, :]
    k_sq = jnp.sum(keys * keys, axis=1)[:, None]
    grid = Q // _BQ
    idx, dist = pl.pallas_call(
        _body,
        grid=(grid,),
        in_specs=[
            pl.BlockSpec((_BQ, D), lambda i: (i, 0)),
            pl.BlockSpec((K, D), lambda i: (0, 0)),
            pl.BlockSpec((1, _BQ), lambda i: (0, i)),
            pl.BlockSpec((K, 1), lambda i: (0, 0)),
        ],
        out_specs=[
            pl.BlockSpec((1, 1, _BQ), lambda i: (i, 0, 0)),
            pl.BlockSpec((1, 1, _BQ), lambda i: (i, 0, 0)),
        ],
        out_shape=[
            jax.ShapeDtypeStruct((grid, 1, _BQ), jnp.int32),
            jax.ShapeDtypeStruct((grid, 1, _BQ), jnp.float32),
        ],
    )(queries, keys, q_sq, k_sq)
    return idx.reshape(Q), dist.reshape(Q)


# speedup vs baseline: 1.5256x; 1.0035x over previous
import jax
import jax.numpy as jnp
from jax.experimental import pallas as pl

_BQ = 4096
_CK = 128


def _body(q_ref, k_ref, qsq_ref, ksq_ref, idx_ref, dist_ref):
    q = q_ref[...]
    k = k_ref[...]
    q_sq = qsq_ref[...]
    k_sq = ksq_ref[...]
    n_keys = k.shape[0]
    runm = runi = None
    for c in range(0, n_keys, _CK):
        dots_c = jax.lax.dot_general(
            k[c:c + _CK], q, (((1,), (1,)), ((), ())),
            preferred_element_type=jnp.float32,
        )
        sc = (q_sq + k_sq[c:c + _CK]) - 2.0 * dots_c
        mc = jnp.min(sc, axis=0)
        iota = jax.lax.broadcasted_iota(jnp.int32, sc.shape, 0)
        ic = jnp.min(jnp.where(sc == mc[None, :], iota, _CK), axis=0) + c
        if runm is None:
            runm, runi = mc, ic
        else:
            better = mc < runm
            runi = jnp.where(better, ic, runi)
            runm = jnp.where(better, mc, runm)
    idx_ref[0, 0, :] = runi
    dist_ref[0, 0, :] = jnp.sqrt(jnp.maximum(runm, 1e-12))


def kernel(queries, keys):
    Q, D = queries.shape
    K, _ = keys.shape
    q_sq = jnp.sum(queries * queries, axis=1)[None, :]
    k_sq = jnp.sum(keys * keys, axis=1)[:, None]
    grid = Q // _BQ
    idx, dist = pl.pallas_call(
        _body,
        grid=(grid,),
        in_specs=[
            pl.BlockSpec((_BQ, D), lambda i: (i, 0)),
            pl.BlockSpec((K, D), lambda i: (0, 0)),
            pl.BlockSpec((1, _BQ), lambda i: (0, i)),
            pl.BlockSpec((K, 1), lambda i: (0, 0)),
        ],
        out_specs=[
            pl.BlockSpec((1, 1, _BQ), lambda i: (i, 0, 0)),
            pl.BlockSpec((1, 1, _BQ), lambda i: (i, 0, 0)),
        ],
        out_shape=[
            jax.ShapeDtypeStruct((grid, 1, _BQ), jnp.int32),
            jax.ShapeDtypeStruct((grid, 1, _BQ), jnp.float32),
        ],
    )(queries, keys, q_sq, k_sq)
    return idx.reshape(Q), dist.reshape(Q)
